# baseline (speedup 1.0000x reference)
import functools

import jax
import jax.numpy as jnp
from jax import lax
from jax.experimental import pallas as pl
from jax.experimental.pallas import tpu as pltpu
from jax.experimental.pallas import tpu_sc as plsc

_NW = 32
_ROWS = 16384
_COLS = 64
_RPW = _ROWS // _NW
_NCHUNK = 4
_CROWS = _RPW // _NCHUNK
_LANES = 16
_UNROLL = 4


def _sc_partials(x):
    mesh = plsc.VectorSubcoreMesh(core_axis_name="c", subcore_axis_name="s")

    @functools.partial(
        pl.kernel,
        mesh=mesh,
        out_type=jax.ShapeDtypeStruct((_NW, _COLS), jnp.float32),
        scratch_types=[
            pltpu.VMEM((_CROWS, _COLS), jnp.float32),
            pltpu.VMEM((_CROWS, _COLS), jnp.float32),
            pltpu.VMEM((_COLS,), jnp.float32),
            pltpu.SemaphoreType.DMA,
            pltpu.SemaphoreType.DMA,
        ],
    )
    def k(x_hbm, out_hbm, buf0, buf1, acc_v, sem0, sem1):
        cid = lax.axis_index("c")
        sid = lax.axis_index("s")
        wid = sid * 2 + cid
        base = wid * _RPW
        bufs = (buf0, buf1)
        sems = (sem0, sem1)

        def copy_of(i):
            return pltpu.make_async_copy(
                x_hbm.at[pl.ds(base + i * _CROWS, _CROWS), :],
                bufs[i % 2],
                sems[i % 2],
            )

        copy_of(0).start()
        accs = (
            jnp.zeros((_LANES,), jnp.float32),
            jnp.zeros((_LANES,), jnp.float32),
            jnp.zeros((_LANES,), jnp.float32),
            jnp.zeros((_LANES,), jnp.float32),
        )
        for i in range(_NCHUNK):
            copy_of(i).wait()
            if i + 1 < _NCHUNK:
                copy_of(i + 1).start()
            buf = bufs[i % 2]

            def body(j, carry, buf=buf):
                a0, a1, a2, a3 = carry
                for u in range(_UNROLL):
                    r = j * _UNROLL + u
                    a0 = a0 + jnp.maximum(buf[r, pl.ds(0, _LANES)], 0.0)
                    a1 = a1 + jnp.maximum(buf[r, pl.ds(16, _LANES)], 0.0)
                    a2 = a2 + jnp.maximum(buf[r, pl.ds(32, _LANES)], 0.0)
                    a3 = a3 + jnp.maximum(buf[r, pl.ds(48, _LANES)], 0.0)
                return a0, a1, a2, a3

            accs = lax.fori_loop(0, _CROWS // _UNROLL, body, accs)

        acc_v[pl.ds(0, _LANES)] = accs[0]
        acc_v[pl.ds(16, _LANES)] = accs[1]
        acc_v[pl.ds(32, _LANES)] = accs[2]
        acc_v[pl.ds(48, _LANES)] = accs[3]
        pltpu.sync_copy(acc_v, out_hbm.at[wid])

    return k(x)


def _combine(p_ref, o_ref):
    s = jnp.sum(p_ref[...], axis=0, keepdims=True)
    o_ref[...] = jnp.sum(s * s, keepdims=True)


def kernel(x):
    partials = _sc_partials(x)
    out = pl.pallas_call(
        _combine,
        out_shape=jax.ShapeDtypeStruct((1, 1), jnp.float32),
    )(partials)
    return out[0, 0]


# speedup vs baseline: 1.0038x; 1.0038x over previous
import functools

import jax
import jax.numpy as jnp
from jax import lax
from jax.experimental import pallas as pl
from jax.experimental.pallas import tpu as pltpu
from jax.experimental.pallas import tpu_sc as plsc

_NW = 32
_ROWS = 16384
_COLS = 64
_RPW = _ROWS // _NW
_NCHUNK = 4
_CROWS = _RPW // _NCHUNK
_LANES = 16
_UNROLL = 4


def _sc_partials(x):
    mesh = plsc.VectorSubcoreMesh(core_axis_name="c", subcore_axis_name="s")

    @functools.partial(
        pl.kernel,
        mesh=mesh,
        out_type=jax.ShapeDtypeStruct((_NW, _COLS), jnp.float32),
        scratch_types=[
            pltpu.VMEM((_CROWS, _COLS), jnp.float32),
            pltpu.VMEM((_CROWS, _COLS), jnp.float32),
            pltpu.VMEM((_COLS,), jnp.float32),
            pltpu.SemaphoreType.DMA,
            pltpu.SemaphoreType.DMA,
        ],
        compiler_params=pltpu.CompilerParams(use_tc_tiling_on_sc=True),
    )
    def k(x_hbm, out_hbm, buf0, buf1, acc_v, sem0, sem1):
        cid = lax.axis_index("c")
        sid = lax.axis_index("s")
        wid = sid * 2 + cid
        base = wid * _RPW
        bufs = (buf0, buf1)
        sems = (sem0, sem1)

        def copy_of(i):
            return pltpu.make_async_copy(
                x_hbm.at[pl.ds(base + i * _CROWS, _CROWS), :],
                bufs[i % 2],
                sems[i % 2],
            )

        copy_of(0).start()
        accs = (
            jnp.zeros((_LANES,), jnp.float32),
            jnp.zeros((_LANES,), jnp.float32),
            jnp.zeros((_LANES,), jnp.float32),
            jnp.zeros((_LANES,), jnp.float32),
        )
        for i in range(_NCHUNK):
            copy_of(i).wait()
            if i + 1 < _NCHUNK:
                copy_of(i + 1).start()
            buf = bufs[i % 2]

            def body(j, carry, buf=buf):
                a0, a1, a2, a3 = carry
                for u in range(_UNROLL):
                    r = j * _UNROLL + u
                    a0 = a0 + jnp.maximum(buf[r, pl.ds(0, _LANES)], 0.0)
                    a1 = a1 + jnp.maximum(buf[r, pl.ds(16, _LANES)], 0.0)
                    a2 = a2 + jnp.maximum(buf[r, pl.ds(32, _LANES)], 0.0)
                    a3 = a3 + jnp.maximum(buf[r, pl.ds(48, _LANES)], 0.0)
                return a0, a1, a2, a3

            accs = lax.fori_loop(0, _CROWS // _UNROLL, body, accs)

        acc_v[pl.ds(0, _LANES)] = accs[0]
        acc_v[pl.ds(16, _LANES)] = accs[1]
        acc_v[pl.ds(32, _LANES)] = accs[2]
        acc_v[pl.ds(48, _LANES)] = accs[3]
        pltpu.sync_copy(acc_v, out_hbm.at[wid])

    return k(x)


def _combine(p_ref, o_ref):
    s = jnp.sum(p_ref[...], axis=0, keepdims=True)
    o_ref[...] = jnp.sum(s * s, keepdims=True)


def kernel(x):
    partials = _sc_partials(x)
    out = pl.pallas_call(
        _combine,
        out_shape=jax.ShapeDtypeStruct((1, 1), jnp.float32),
    )(partials)
    return out[0, 0]


# speedup vs baseline: 1.2266x; 1.2220x over previous
import functools

import jax
import jax.numpy as jnp
from jax import lax
from jax.experimental import pallas as pl
from jax.experimental.pallas import tpu as pltpu
from jax.experimental.pallas import tpu_sc as plsc

_NW = 32
_ROWS = 64
_COLS = 16384
_RPW = _ROWS // _NW
_NCHUNK = 4
_CW = _COLS // _NCHUNK
_LANES = 16
_UNROLL = 4


def _sc_partials(xt):
    mesh = plsc.VectorSubcoreMesh(core_axis_name="c", subcore_axis_name="s")

    @functools.partial(
        pl.kernel,
        mesh=mesh,
        out_type=jax.ShapeDtypeStruct((_ROWS, _LANES), jnp.float32),
        scratch_types=[
            pltpu.VMEM((_RPW, _CW), jnp.float32),
            pltpu.VMEM((_RPW, _CW), jnp.float32),
            pltpu.VMEM((_RPW, _LANES), jnp.float32),
            pltpu.SemaphoreType.DMA,
            pltpu.SemaphoreType.DMA,
        ],
    )
    def k(xt_hbm, out_hbm, buf0, buf1, accrow, sem0, sem1):
        cid = lax.axis_index("c")
        sid = lax.axis_index("s")
        wid = sid * 2 + cid
        r0 = wid * _RPW
        bufs = (buf0, buf1)
        sems = (sem0, sem1)

        def copy_of(i):
            return pltpu.make_async_copy(
                xt_hbm.at[pl.ds(r0, _RPW), pl.ds(i * _CW, _CW)],
                bufs[i % 2],
                sems[i % 2],
            )

        copy_of(0).start()
        accs = (
            jnp.zeros((_LANES,), jnp.float32),
            jnp.zeros((_LANES,), jnp.float32),
        )
        for i in range(_NCHUNK):
            copy_of(i).wait()
            if i + 1 < _NCHUNK:
                copy_of(i + 1).start()
            buf = bufs[i % 2]

            def body(j, carry, buf=buf):
                a0, a1 = carry
                for u in range(_UNROLL):
                    col = (j * _UNROLL + u) * _LANES
                    a0 = a0 + jnp.maximum(buf[0, pl.ds(col, _LANES)], 0.0)
                    a1 = a1 + jnp.maximum(buf[1, pl.ds(col, _LANES)], 0.0)
                return a0, a1

            accs = lax.fori_loop(0, _CW // (_LANES * _UNROLL), body, accs)

        accrow[0, :] = accs[0]
        accrow[1, :] = accs[1]
        pltpu.sync_copy(accrow, out_hbm.at[pl.ds(r0, _RPW)])

    return k(xt)


def _combine(p_ref, o_ref):
    s = jnp.sum(p_ref[...], axis=1, keepdims=True)
    o_ref[...] = jnp.sum(s * s, keepdims=True)


def kernel(x):
    xt = x.T
    partials = _sc_partials(xt)
    out = pl.pallas_call(
        _combine,
        out_shape=jax.ShapeDtypeStruct((1, 1), jnp.float32),
    )(partials)
    return out[0, 0]
